# baseline (speedup 1.0000x reference)
import jax
import jax.numpy as jnp
from jax.experimental import pallas as pl
from jax.experimental.pallas import tpu as pltpu

_M1, _M2, _A1, _A2, _T = 2.0, 2.0, 2.4, 2.2, 1.0
_N = 4096
_D = 128
_K = _D + 2
_BM = 256
_NC = 256
_LOG2E = 1.4426950408889634


def _dot(a, b, dims):
    return jax.lax.dot_general(a, b, (dims, ((), ())),
                               preferred_element_type=jnp.float32)


def _rowpick(mat, onehot):
    return jnp.sum(mat * onehot, axis=1, keepdims=True)


def _loss_kernel(s_row_ref, t_col_ref, s_col_ref, x_ref,
                 dist_ref, loss_ref, lhs_ref, rhs_ref, cnt_ref, s2_ref,
                 kh_ref, khf_ref, acc_ref):
    i = pl.program_id(0)
    nblocks = pl.num_programs(0)

    @pl.when(i == 0)
    def _init():
        xr = x_ref[...]
        sq = jnp.sum(xr * xr, axis=1, keepdims=True)
        inv = 1.0 / jnp.maximum(jnp.sqrt(sq), 1e-12)
        xn = xr * inv
        xq = jnp.sum(xn * xn, axis=1, keepdims=True)
        one_col = jnp.ones((_N, 1), dtype=jnp.float32)
        lhs_ref[:, 0:_D] = xn
        lhs_ref[:, _D:_D + 1] = one_col
        lhs_ref[:, _D + 1:_K] = xq
        rhs_ref[:, 0:_D] = xn * (-2.0)
        rhs_ref[:, _D:_D + 1] = xq
        rhs_ref[:, _D + 1:_K] = one_col

        tc = t_col_ref[...]
        sc = s_col_ref[...]
        kiota = jax.lax.broadcasted_iota(jnp.int32, (1, _NC), 1)
        key = tc * 2 + sc
        kh = (key == kiota).astype(jnp.float32)
        khf = ((key - sc * 2 + 1) == kiota).astype(jnp.float32)
        kh_ref[...] = kh
        khf_ref[...] = khf
        s2_ref[...] = (sc == jax.lax.broadcasted_iota(jnp.int32, (1, 2), 1)
                       ).astype(jnp.float32)
        ones_n = jnp.ones((1, _N), dtype=jnp.float32)
        cpair = _dot(ones_n, kh, ((1,), (0,)))
        paircnt = _dot(kh, cpair, ((1,), (1,)))
        crosscnt = _dot(khf, cpair, ((1,), (1,)))
        cnt_ref[:, 0:1] = paircnt - 1.0
        cnt_ref[:, 1:2] = crosscnt
        acc_ref[0, 0] = 0.0

    r0 = i * _BM
    lhs_blk = lhs_ref[pl.ds(r0, _BM), :]
    d2 = _dot(lhs_blk, rhs_ref[...], ((1,), (1,)))
    m = jnp.maximum(d2, 1e-12)
    dist = m * jax.lax.rsqrt(m)
    dist_ref[...] = dist

    s_row = s_row_ref[...]
    scb = s_col_ref[pl.ds(r0, _BM), :]

    _DA = _A1 - _A2
    sf_row = s_row.astype(jnp.float32)
    sf_col = scb.astype(jnp.float32)
    c1 = _A1 - _DA * sf_row
    c2 = (2.0 * _DA) * sf_row - _DA
    alpha = c1 + sf_col * c2
    u = alpha - dist

    apv = jnp.maximum(_M1 - u, 0.0)
    e = jax.lax.exp2(u * _LOG2E)
    w = jnp.where(u > 0.0, e, 0.0)
    num = u * w

    s2 = s2_ref[...]
    w2 = _dot(w, s2, ((1,), (0,)))
    n2 = _dot(num, s2, ((1,), (0,)))
    akh = _dot(apv, kh_ref[...], ((1,), (0,)))
    wkh = _dot(w, kh_ref[...], ((1,), (0,)))
    nkh = _dot(num, kh_ref[...], ((1,), (0,)))

    khb = kh_ref[pl.ds(r0, _BM), :]
    khfb = khf_ref[pl.ds(r0, _BM), :]
    ap_i = _rowpick(akh, khb)
    ap_c = _rowpick(akh, khfb)
    sint = scb == 0
    wg_i = jnp.where(sint, w2[:, 0:1], w2[:, 1:2])
    wg_c = (w2[:, 0:1] + w2[:, 1:2]) - wg_i
    ng_i = jnp.where(sint, n2[:, 0:1], n2[:, 1:2])
    ng_c = (n2[:, 0:1] + n2[:, 1:2]) - ng_i
    w_i = wg_i - _rowpick(wkh, khb)
    w_c = wg_c - _rowpick(wkh, khfb)
    n_i = ng_i - _rowpick(nkh, khb)
    n_c = ng_c - _rowpick(nkh, khfb)

    cnt = cnt_ref[pl.ds(r0, _BM), :]
    row_loss = (ap_i / (cnt[:, 0:1] + 1e-5) + ap_c / (cnt[:, 1:2] + 1e-5)
                + n_i / (w_i + 1e-5) + n_c / (w_c + 1e-5))
    acc_ref[0, 0] += jnp.sum(row_loss)

    @pl.when(i == nblocks - 1)
    def _final():
        loss_ref[...] = jnp.full((1, 1), acc_ref[0, 0] / jnp.float32(_N),
                                 dtype=jnp.float32)


@jax.jit
def kernel(x, targets, sub):
    s_row = sub.reshape(1, _N).astype(jnp.int32)
    t_col = targets.reshape(_N, 1).astype(jnp.int32)
    s_col = sub.reshape(_N, 1).astype(jnp.int32)

    grid = (_N // _BM,)
    dist, loss = pl.pallas_call(
        _loss_kernel,
        grid=grid,
        in_specs=[
            pl.BlockSpec((1, _N), lambda i: (0, 0)),
            pl.BlockSpec((_N, 1), lambda i: (0, 0)),
            pl.BlockSpec((_N, 1), lambda i: (0, 0)),
            pl.BlockSpec((_N, _D), lambda i: (0, 0)),
        ],
        out_specs=[
            pl.BlockSpec((_BM, _N), lambda i: (i, 0)),
            pl.BlockSpec((1, 1), lambda i: (0, 0)),
        ],
        out_shape=[
            jax.ShapeDtypeStruct((_N, _N), jnp.float32),
            jax.ShapeDtypeStruct((1, 1), jnp.float32),
        ],
        scratch_shapes=[
            pltpu.VMEM((_N, _K), jnp.float32),
            pltpu.VMEM((_N, _K), jnp.float32),
            pltpu.VMEM((_N, 2), jnp.float32),
            pltpu.VMEM((_N, 2), jnp.float32),
            pltpu.VMEM((_N, _NC), jnp.float32),
            pltpu.VMEM((_N, _NC), jnp.float32),
            pltpu.SMEM((1, 1), jnp.float32),
        ],
    )(s_row, t_col, s_col, x)
    return loss.reshape(()), dist


# speedup vs baseline: 1.3989x; 1.3989x over previous
import jax
import jax.numpy as jnp
from jax.experimental import pallas as pl
from jax.experimental.pallas import tpu as pltpu

_M1, _M2, _A1, _A2, _T = 2.0, 2.0, 2.4, 2.2, 1.0
_N = 4096
_D = 128
_K = _D + 2
_BM = 512
_NC = 256
_LOG2E = 1.4426950408889634


def _dot(a, b, dims):
    return jax.lax.dot_general(a, b, (dims, ((), ())),
                               preferred_element_type=jnp.float32)


def _rowsum(mat):
    return jnp.sum(mat, axis=1, keepdims=True)


def _loss_kernel(s_row_ref, key_ref, x_ref,
                 dist_ref, loss_ref, lhs_ref, rhs_ref, kh_ref, cp_ref,
                 acc_ref):
    i = pl.program_id(0)
    nblocks = pl.num_programs(0)
    kiota = jax.lax.broadcasted_iota(jnp.int32, (1, _NC), 1)

    @pl.when(i == 0)
    def _init():
        xr = x_ref[...]
        sq = jnp.sum(xr * xr, axis=1, keepdims=True)
        inv = 1.0 / jnp.maximum(jnp.sqrt(sq), 1e-12)
        xn = xr * inv
        xq = jnp.sum(xn * xn, axis=1, keepdims=True)
        one_col = jnp.ones((_N, 1), dtype=jnp.float32)
        lhs_ref[:, 0:_D] = xn
        lhs_ref[:, _D:_D + 1] = one_col
        lhs_ref[:, _D + 1:_K] = xq
        rhs_ref[:, 0:_D] = xn * (-2.0)
        rhs_ref[:, _D:_D + 1] = xq
        rhs_ref[:, _D + 1:_K] = one_col

        kh = (key_ref[...] == kiota).astype(jnp.float32)
        kh_ref[...] = kh
        ones_n = jnp.ones((1, _N), dtype=jnp.float32)
        cp_ref[...] = _dot(ones_n, kh, ((1,), (0,)))
        acc_ref[0, 0] = 0.0

    r0 = i * _BM
    lhs_blk = lhs_ref[pl.ds(r0, _BM), :]
    d2 = _dot(lhs_blk, rhs_ref[...], ((1,), (1,)))
    m = jnp.maximum(d2, 1e-12)
    dist = m * jax.lax.rsqrt(m)
    dist_ref[...] = dist

    keyb = key_ref[pl.ds(r0, _BM), :]
    scb = jnp.bitwise_and(keyb, 1)

    _DA = _A1 - _A2
    sf_row = s_row_ref[...].astype(jnp.float32)
    sf_col = scb.astype(jnp.float32)
    c1 = _A1 - _DA * sf_row
    c2 = (2.0 * _DA) * sf_row - _DA
    alpha = c1 + sf_col * c2
    u = alpha - dist

    apv = jnp.maximum(_M1 - u, 0.0)
    e = jax.lax.exp2(u * _LOG2E)
    w = jnp.where(u > 0.0, e, 0.0)
    num = u * w

    akh = _dot(apv, kh_ref[...], ((1,), (0,)))
    wkh = _dot(w, kh_ref[...], ((1,), (0,)))
    nkh = _dot(num, kh_ref[...], ((1,), (0,)))

    khb = (keyb == kiota).astype(jnp.float32)
    khfb = (jnp.bitwise_xor(keyb, 1) == kiota).astype(jnp.float32)
    parb = (jnp.bitwise_and(kiota, 1) == scb).astype(jnp.float32)

    cp = cp_ref[...]
    cnt_i = _rowsum(cp * khb) - 1.0
    cnt_c = _rowsum(cp * khfb)
    ap_i = _rowsum(akh * khb)
    ap_c = _rowsum(akh * khfb)
    wg_i = _rowsum(wkh * parb)
    wg_c = _rowsum(wkh) - wg_i
    ng_i = _rowsum(nkh * parb)
    ng_c = _rowsum(nkh) - ng_i
    w_i = wg_i - _rowsum(wkh * khb)
    w_c = wg_c - _rowsum(wkh * khfb)
    n_i = ng_i - _rowsum(nkh * khb)
    n_c = ng_c - _rowsum(nkh * khfb)

    row_loss = (ap_i / (cnt_i + 1e-5) + ap_c / (cnt_c + 1e-5)
                + n_i / (w_i + 1e-5) + n_c / (w_c + 1e-5))
    acc_ref[0, 0] += jnp.sum(row_loss)

    @pl.when(i == nblocks - 1)
    def _final():
        loss_ref[...] = jnp.full((1, 1), acc_ref[0, 0] / jnp.float32(_N),
                                 dtype=jnp.float32)


@jax.jit
def kernel(x, targets, sub):
    s_row = sub.reshape(1, _N).astype(jnp.int32)
    key_col = (targets.astype(jnp.int32) * 2
               + sub.astype(jnp.int32)).reshape(_N, 1)

    grid = (_N // _BM,)
    dist, loss = pl.pallas_call(
        _loss_kernel,
        grid=grid,
        compiler_params=pltpu.CompilerParams(
            vmem_limit_bytes=100 * 1024 * 1024),
        in_specs=[
            pl.BlockSpec((1, _N), lambda i: (0, 0)),
            pl.BlockSpec((_N, 1), lambda i: (0, 0)),
            pl.BlockSpec((_N, _D), lambda i: (0, 0)),
        ],
        out_specs=[
            pl.BlockSpec((_BM, _N), lambda i: (i, 0)),
            pl.BlockSpec((1, 1), lambda i: (0, 0)),
        ],
        out_shape=[
            jax.ShapeDtypeStruct((_N, _N), jnp.float32),
            jax.ShapeDtypeStruct((1, 1), jnp.float32),
        ],
        scratch_shapes=[
            pltpu.VMEM((_N, _K), jnp.float32),
            pltpu.VMEM((_N, _K), jnp.float32),
            pltpu.VMEM((_N, _NC), jnp.float32),
            pltpu.VMEM((1, _NC), jnp.float32),
            pltpu.SMEM((1, 1), jnp.float32),
        ],
    )(s_row, key_col, x)
    return loss.reshape(()), dist


# speedup vs baseline: 1.4585x; 1.0425x over previous
import jax
import jax.numpy as jnp
from jax.experimental import pallas as pl
from jax.experimental.pallas import tpu as pltpu

_M1, _M2, _A1, _A2, _T = 2.0, 2.0, 2.4, 2.2, 1.0
_N = 4096
_D = 128
_K = _D + 2
_BM = 512
_NC = 256
_LOG2E = 1.4426950408889634


def _dot(a, b, dims):
    return jax.lax.dot_general(a, b, (dims, ((), ())),
                               preferred_element_type=jnp.float32)


def _rowsum(mat):
    return jnp.sum(mat, axis=1, keepdims=True)


def _loss_kernel(s_row_ref, key_ref, x_ref,
                 dist_ref, loss_ref, lhs_ref, rhs_ref, kh_ref, cp_ref,
                 acc_ref):
    i = pl.program_id(0)
    nblocks = pl.num_programs(0)
    kiota = jax.lax.broadcasted_iota(jnp.int32, (1, _NC), 1)

    @pl.when(i == 0)
    def _init():
        xr = x_ref[...]
        sq = jnp.sum(xr * xr, axis=1, keepdims=True)
        inv = 1.0 / jnp.maximum(jnp.sqrt(sq), 1e-12)
        xn = xr * inv
        xq = jnp.sum(xn * xn, axis=1, keepdims=True)
        one_col = jnp.ones((_N, 1), dtype=jnp.float32)
        lhs_ref[:, 0:_D] = xn
        lhs_ref[:, _D:_D + 1] = one_col
        lhs_ref[:, _D + 1:_K] = xq
        rhs_ref[:, 0:_D] = xn * (-2.0)
        rhs_ref[:, _D:_D + 1] = xq
        rhs_ref[:, _D + 1:_K] = one_col

        kh = (key_ref[...] == kiota).astype(jnp.float32)
        kh_ref[...] = kh
        ones_n = jnp.ones((1, _N), dtype=jnp.float32)
        cp_ref[...] = _dot(ones_n, kh, ((1,), (0,)))
        acc_ref[0, 0] = 0.0

    r0 = i * _BM
    lhs_blk = lhs_ref[pl.ds(r0, _BM), :]
    d2 = _dot(lhs_blk, rhs_ref[...], ((1,), (1,)))
    m = jnp.maximum(d2, 1e-12)
    dist = m * jax.lax.rsqrt(m)
    dist_ref[...] = dist

    keyb = key_ref[pl.ds(r0, _BM), :]
    scb = jnp.bitwise_and(keyb, 1)

    _B1, _B2 = _A1 - _M1, _A2 - _M1
    _DB = _B1 - _B2
    sf_row = s_row_ref[...].astype(jnp.float32)
    sf_col = scb.astype(jnp.float32)
    c1 = _B1 - _DB * sf_row
    c2 = (2.0 * _DB) * sf_row - _DB
    beta = c1 + sf_col * c2

    apv = jnp.maximum(dist - beta, 0.0)
    wb = jax.lax.exp2(dist * (-_LOG2E))
    v = dist * wb

    akh = _dot(apv, kh_ref[...], ((1,), (0,)))
    wkh = _dot(wb, kh_ref[...], ((1,), (0,)))
    vkh = _dot(v, kh_ref[...], ((1,), (0,)))

    khb = (keyb == kiota).astype(jnp.float32)
    khfb = (jnp.bitwise_xor(keyb, 1) == kiota).astype(jnp.float32)
    parb = (jnp.bitwise_and(kiota, 1) == scb).astype(jnp.float32)

    cp = cp_ref[...]
    cnt_i = _rowsum(cp * khb) - 1.0
    cnt_c = _rowsum(cp * khfb)
    ap_i = _rowsum(akh * khb)
    ap_c = _rowsum(akh * khfb)
    wb_same = _rowsum(wkh * khb)
    wb_fsame = _rowsum(wkh * khfb)
    wb_i = _rowsum(wkh * parb) - wb_same
    wb_c = _rowsum(wkh) - wb_i - wb_same - wb_fsame
    v_same = _rowsum(vkh * khb)
    v_fsame = _rowsum(vkh * khfb)
    v_i = _rowsum(vkh * parb) - v_same
    v_c = _rowsum(vkh) - v_i - v_same - v_fsame
    _EA1, _EA2 = 11.023176380641601, 9.025013499434122
    w_i = _EA1 * wb_i
    w_c = _EA2 * wb_c
    n_i = _EA1 * (_A1 * wb_i - v_i)
    n_c = _EA2 * (_A2 * wb_c - v_c)

    row_loss = (ap_i / (cnt_i + 1e-5) + ap_c / (cnt_c + 1e-5)
                + n_i / (w_i + 1e-5) + n_c / (w_c + 1e-5))
    acc_ref[0, 0] += jnp.sum(row_loss)

    @pl.when(i == nblocks - 1)
    def _final():
        loss_ref[...] = jnp.full((1, 1), acc_ref[0, 0] / jnp.float32(_N),
                                 dtype=jnp.float32)


@jax.jit
def kernel(x, targets, sub):
    s_row = sub.reshape(1, _N).astype(jnp.int32)
    key_col = (targets.astype(jnp.int32) * 2
               + sub.astype(jnp.int32)).reshape(_N, 1)

    grid = (_N // _BM,)
    dist, loss = pl.pallas_call(
        _loss_kernel,
        grid=grid,
        compiler_params=pltpu.CompilerParams(
            vmem_limit_bytes=100 * 1024 * 1024),
        in_specs=[
            pl.BlockSpec((1, _N), lambda i: (0, 0)),
            pl.BlockSpec((_N, 1), lambda i: (0, 0)),
            pl.BlockSpec((_N, _D), lambda i: (0, 0)),
        ],
        out_specs=[
            pl.BlockSpec((_BM, _N), lambda i: (i, 0)),
            pl.BlockSpec((1, 1), lambda i: (0, 0)),
        ],
        out_shape=[
            jax.ShapeDtypeStruct((_N, _N), jnp.float32),
            jax.ShapeDtypeStruct((1, 1), jnp.float32),
        ],
        scratch_shapes=[
            pltpu.VMEM((_N, _K), jnp.float32),
            pltpu.VMEM((_N, _K), jnp.float32),
            pltpu.VMEM((_N, _NC), jnp.float32),
            pltpu.VMEM((1, _NC), jnp.float32),
            pltpu.SMEM((1, 1), jnp.float32),
        ],
    )(s_row, key_col, x)
    return loss.reshape(()), dist
